# baseline (speedup 1.0000x reference)
import functools

import jax
import jax.numpy as jnp
from jax import lax
from jax.experimental import pallas as pl
from jax.experimental.pallas import tpu as pltpu
from jax.experimental.pallas import tpu_sc as plsc

_LANES = 16


def _gather_bias(ind_pairs_pad, table_flat, num_heads, side, area_pad,
                 per_tile):
    info = plsc.get_sparse_core_info()
    nc = info.num_cores
    chunks = per_tile // _LANES
    tbl_stride = side * side
    mesh = plsc.VectorSubcoreMesh(core_axis_name="c", subcore_axis_name="s")

    @functools.partial(
        pl.kernel,
        out_type=jax.ShapeDtypeStruct((num_heads * area_pad,), jnp.float32),
        mesh=mesh,
        compiler_params=pltpu.CompilerParams(needs_layout_passes=False),
        scratch_types=[
            pltpu.VMEM((2 * per_tile,), jnp.int32),
            pltpu.VMEM((num_heads * per_tile,), jnp.float32),
            pltpu.VMEM((table_flat.shape[0],), jnp.float32),
        ],
    )
    def k(ind_hbm, tbl_hbm, out_hbm, ind_v, out_v, tbl_v):
        wid = lax.axis_index("s") * nc + lax.axis_index("c")
        base = wid * per_tile
        pltpu.sync_copy(ind_hbm.at[pl.ds(base * 2, 2 * per_tile)], ind_v)
        pltpu.sync_copy(tbl_hbm, tbl_v)
        lane = lax.iota(jnp.int32, _LANES)

        def chunk(c, carry):
            p0 = c * (2 * _LANES) + 2 * lane
            i0 = plsc.load_gather(ind_v, [p0])
            i1 = plsc.load_gather(ind_v, [p0 + 1])
            flat = i0 * side + i1
            for h in range(num_heads):
                vals = plsc.load_gather(tbl_v, [flat + h * tbl_stride])
                out_v[pl.ds(h * per_tile + c * _LANES, _LANES)] = vals
            return carry

        lax.fori_loop(0, chunks, chunk, 0)
        for h in range(num_heads):
            pltpu.sync_copy(out_v.at[pl.ds(h * per_tile, per_tile)],
                            out_hbm.at[pl.ds(h * area_pad + base, per_tile)])

    return k(ind_pairs_pad, table_flat).reshape(num_heads, area_pad)


def _add_body(a_ref, b_ref, o_ref):
    o_ref[...] = a_ref[...] + b_ref[None]


def kernel(attn, rel_pos_table, rel_pos_ind):
    nw, nh, a1, a2 = attn.shape
    area = a1 * a2
    side = rel_pos_table.shape[2]

    n_tiles = 32
    per_tile = -(-area // (n_tiles * _LANES)) * _LANES
    area_pad = n_tiles * per_tile

    ind_pairs = rel_pos_ind.reshape(-1).astype(jnp.int32)
    ind_pairs_pad = jnp.pad(ind_pairs, (0, 2 * area_pad - ind_pairs.shape[0]))
    table_flat = rel_pos_table.reshape(-1)

    bias_pad = _gather_bias(ind_pairs_pad, table_flat, nh, side,
                            area_pad, per_tile)
    bias = bias_pad[:, :area]

    attn3 = attn.reshape(nw, nh, area)
    out3 = pl.pallas_call(
        _add_body,
        grid=(nw,),
        in_specs=[
            pl.BlockSpec((1, nh, area), lambda w: (w, 0, 0)),
            pl.BlockSpec((nh, area), lambda w: (0, 0)),
        ],
        out_specs=pl.BlockSpec((1, nh, area), lambda w: (w, 0, 0)),
        out_shape=jax.ShapeDtypeStruct((nw, nh, area), jnp.float32),
    )(attn3, bias)
    return out3.reshape(attn.shape)


# speedup vs baseline: 1.0046x; 1.0046x over previous
import functools

import jax
import jax.numpy as jnp
from jax import lax
from jax.experimental import pallas as pl
from jax.experimental.pallas import tpu as pltpu
from jax.experimental.pallas import tpu_sc as plsc

_LANES = 16


def _gather_bias(ind_pairs_pad, table_flat, num_heads, side, area_pad,
                 per_tile):
    info = plsc.get_sparse_core_info()
    nc = info.num_cores
    chunks = per_tile // _LANES
    tbl_stride = side * side
    mesh = plsc.VectorSubcoreMesh(core_axis_name="c", subcore_axis_name="s")

    @functools.partial(
        pl.kernel,
        out_type=jax.ShapeDtypeStruct((num_heads * area_pad,), jnp.float32),
        mesh=mesh,
        compiler_params=pltpu.CompilerParams(needs_layout_passes=False),
        scratch_types=[
            pltpu.VMEM((2 * per_tile,), jnp.int32),
            pltpu.VMEM((num_heads * per_tile,), jnp.float32),
            pltpu.VMEM((table_flat.shape[0],), jnp.float32),
        ],
    )
    def k(ind_hbm, tbl_hbm, out_hbm, ind_v, out_v, tbl_v):
        wid = lax.axis_index("s") * nc + lax.axis_index("c")
        base = wid * per_tile
        pltpu.sync_copy(ind_hbm.at[pl.ds(base * 2, 2 * per_tile)], ind_v)
        pltpu.sync_copy(tbl_hbm, tbl_v)
        lane = lax.iota(jnp.int32, _LANES)

        def chunk(c, carry):
            p0 = c * (2 * _LANES) + 2 * lane
            i0 = plsc.load_gather(ind_v, [p0])
            i1 = plsc.load_gather(ind_v, [p0 + 1])
            flat = i0 * side + i1
            for h in range(num_heads):
                vals = plsc.load_gather(tbl_v, [flat + h * tbl_stride])
                out_v[pl.ds(h * per_tile + c * _LANES, _LANES)] = vals
            return carry

        lax.fori_loop(0, chunks, chunk, 0)
        for h in range(num_heads):
            pltpu.sync_copy(out_v.at[pl.ds(h * per_tile, per_tile)],
                            out_hbm.at[pl.ds(h * area_pad + base, per_tile)])

    return k(ind_pairs_pad, table_flat).reshape(num_heads, area_pad)


def _add_body2(a_ref, b_ref, o_ref):
    o_ref[...] = a_ref[...] + b_ref[...]


def kernel(attn, rel_pos_table, rel_pos_ind):
    nw, nh, a1, a2 = attn.shape
    area = a1 * a2
    side = rel_pos_table.shape[2]

    n_tiles = 32
    per_tile = -(-area // (n_tiles * _LANES)) * _LANES
    area_pad = n_tiles * per_tile

    ind_pairs = rel_pos_ind.reshape(-1).astype(jnp.int32)
    ind_pairs_pad = jnp.pad(ind_pairs, (0, 2 * area_pad - ind_pairs.shape[0]))
    table_flat = rel_pos_table.reshape(-1)

    bias_pad = _gather_bias(ind_pairs_pad, table_flat, nh, side,
                            area_pad, per_tile)

    wpb = 4
    rows = nh * wpb
    bias_blk = jnp.tile(bias_pad[:, :area], (wpb, 1))

    attn2 = attn.reshape(nw * nh, area)
    out2 = pl.pallas_call(
        _add_body2,
        grid=(nw // wpb,),
        in_specs=[
            pl.BlockSpec((rows, area), lambda w: (w, 0)),
            pl.BlockSpec((rows, area), lambda w: (0, 0)),
        ],
        out_specs=pl.BlockSpec((rows, area), lambda w: (w, 0)),
        out_shape=jax.ShapeDtypeStruct((nw * nh, area), jnp.float32),
    )(attn2, bias_blk)
    return out2.reshape(attn.shape)
